# baseline (speedup 1.0000x reference)
import functools

import jax
import jax.numpy as jnp
from jax import lax
from jax.experimental import pallas as pl
from jax.experimental.pallas import tpu as pltpu

CUTOFF = 5.0
G = 50
H = 128
NUM_GRAPHS = 256
TI = 128
TIL = 32
TJ = 128


def _embed_kernel(z_ref, emb1_ref, emb2_ref, emb3_ref, emb4_ref, W_ref, b_ref,
                  v_ref):
    z = z_ref[...]

    def onehot(col, k):
        ids = z[:, col][:, None]
        return (ids == lax.broadcasted_iota(jnp.int32, (TI, k), 1)).astype(
            jnp.float32)

    e1 = jnp.dot(onehot(0, 100), emb1_ref[...],
                 preferred_element_type=jnp.float32)
    e2 = jnp.dot(onehot(1, 10), emb2_ref[...],
                 preferred_element_type=jnp.float32)
    e3 = jnp.dot(onehot(2, 10), emb3_ref[...],
                 preferred_element_type=jnp.float32)
    e4 = jnp.dot(onehot(3, 10), emb4_ref[...],
                 preferred_element_type=jnp.float32)
    W = W_ref[...]
    v = (jnp.dot(e1, W[:, 0:H].T, preferred_element_type=jnp.float32)
         + jnp.dot(e2, W[:, H:2 * H].T, preferred_element_type=jnp.float32)
         + jnp.dot(e3, W[:, 2 * H:3 * H].T, preferred_element_type=jnp.float32)
         + jnp.dot(e4, W[:, 3 * H:4 * H].T, preferred_element_type=jnp.float32)
         + b_ref[...])
    v_ref[...] = v


def _layer_kernel(jstart8_ref, jnum_ref, feati_ref, featj_ref, v_ref, vl_ref,
                  W1_ref, b1_ref, W2_ref, b2_ref, v1W_ref, v1b_ref, v2W_ref,
                  v2b_ref, vout_ref):
    t = pl.program_id(0)
    i0 = t * TIL
    iidx = i0 + lax.broadcasted_iota(jnp.int32, (TIL, 1), 0)

    step = CUTOFF / (G - 1)
    coeff = -0.5 / (step * step)
    offset = step * lax.broadcasted_iota(jnp.int32, (1, 1, G), 2).astype(
        jnp.float32)
    sg = -2.0 * coeff * offset
    tg = coeff * offset * offset

    W1t = W1_ref[...].T
    W2t = W2_ref[...].T
    b1 = b1_ref[...]
    b2 = b2_ref[...]
    jbase = jstart8_ref[t] * 8
    feati = feati_ref[...]
    fi_pos = feati[:, 0:5]
    fi_b = feati[:, 5:8]

    def body(jt, acc):
        j0 = jbase + jt * TJ
        featj = featj_ref[pl.ds(j0, TJ), :]
        vlj = vl_ref[pl.ds(j0, TJ), :]

        cn = (((1,), (1,)), ((), ()))
        d2 = lax.dot_general(fi_pos, featj[:, 0:5], cn,
                             precision=lax.Precision.HIGHEST,
                             preferred_element_type=jnp.float32)
        db2 = lax.dot_general(fi_b, featj[:, 5:8], cn,
                              precision=lax.Precision.HIGHEST,
                              preferred_element_type=jnp.float32)
        d2 = jnp.maximum(d2, 0.0)
        d = jnp.sqrt(d2)
        jidx = j0 + lax.broadcasted_iota(jnp.int32, (1, TJ), 1)
        mask = (d < CUTOFF) & (db2 < 0.5) & (iidx != jidx)
        C = 0.5 * (jnp.cos(d * (jnp.pi / CUTOFF)) + 1.0)
        scal = jnp.where(mask, C, 0.0)

        q = coeff * d2
        de = jnp.exp(q[:, :, None] + d[:, :, None] * sg + tg)
        de_f = de.reshape(TIL * TJ, G)
        h = jnp.maximum(
            jnp.dot(de_f, W1t, preferred_element_type=jnp.float32) + b1, 0.0)
        wg = jnp.dot(h, W2t, preferred_element_type=jnp.float32) + b2
        wg3 = wg.reshape(TIL, TJ, H)
        contrib = wg3 * (scal[:, :, None] * vlj[None, :, :])
        return acc + jnp.sum(contrib, axis=1)

    acc = lax.fori_loop(0, jnum_ref[t], body,
                        jnp.zeros((TIL, H), jnp.float32))

    vi = v_ref[...]
    o = jnp.maximum(
        jnp.dot(acc, v1W_ref[...].T, preferred_element_type=jnp.float32)
        + v1b_ref[...], 0.0)
    o = jnp.dot(o, v2W_ref[...].T, preferred_element_type=jnp.float32) \
        + v2b_ref[...]
    vout_ref[...] = vi + o


def _vl_kernel(v_ref, W_ref, out_ref):
    out_ref[...] = jnp.dot(v_ref[...], W_ref[...].T,
                           preferred_element_type=jnp.float32)


def _final_kernel(x1_ref, x2_ref, x3_ref, batchT_ref, W1_ref, b1_ref, W2_ref,
                  b2_ref, num_tiles_ref, out_ref, acc_ref, *, num_graphs):
    t = pl.program_id(0)

    @pl.when(t == 0)
    def _():
        acc_ref[...] = jnp.zeros_like(acc_ref)

    W1 = W1_ref[...]
    u = (jnp.dot(x1_ref[...], W1[:, 0:H].T, preferred_element_type=jnp.float32)
         + jnp.dot(x2_ref[...], W1[:, H:2 * H].T,
                   preferred_element_type=jnp.float32)
         + jnp.dot(x3_ref[...], W1[:, 2 * H:3 * H].T,
                   preferred_element_type=jnp.float32)
         + b1_ref[...])
    u = jnp.maximum(u, 0.0)

    bj = batchT_ref[:, pl.ds(t * TI, TI)]
    ohT = (lax.broadcasted_iota(jnp.int32, (num_graphs, TI), 0) == bj).astype(
        jnp.float32)
    acc_ref[...] += jnp.dot(ohT, u, preferred_element_type=jnp.float32)

    @pl.when(t == num_tiles_ref[0] - 1)
    def _():
        out_ref[...] = jnp.maximum(
            jnp.dot(acc_ref[...], W2_ref[...].T,
                    preferred_element_type=jnp.float32) + b2_ref[...], 0.0)


def _forward(z, pos, batch, emb1, emb2, emb3, emb4, lin1_W, lin1_b, e_mlp1_W,
             e_mlp1_b, e_mlp2_W, e_mlp2_b, e_lin_W, v_lin1_W, v_lin1_b,
             v_lin2_W, v_lin2_b, u_lin1_W, u_lin1_b, u_lin2_W, u_lin2_b,
             num_graphs, interpret=False):
    n = pos.shape[0]
    num_tiles = (n + TI - 1) // TI
    npad = num_tiles * TI
    pad = npad - n

    batch = batch.astype(jnp.int32)
    z = z.astype(jnp.int32)
    pos_p = jnp.pad(pos, ((0, pad), (0, 0)))
    batch_p = jnp.pad(batch, (0, pad), constant_values=jnp.int32(2 ** 30))
    z_p = jnp.pad(z, ((0, pad), (0, 0)))
    batchT = batch_p[None, :]

    num_itiles = npad // TIL
    tstarts = jnp.arange(num_itiles, dtype=jnp.int32) * TIL
    row_lo = jnp.clip(tstarts, 0, n - 1)
    row_hi = jnp.clip(tstarts + TIL - 1, 0, n - 1)
    blo = batch[row_lo]
    bhi = batch[row_hi]
    jlo = jnp.searchsorted(batch, blo, side="left").astype(jnp.int32)
    jhi = jnp.searchsorted(batch, bhi, side="right").astype(jnp.int32)
    jstart8 = jlo // 8
    jnum = (jhi - jstart8 * 8 + TJ - 1) // TJ

    x, y, zc = pos_p[:, 0], pos_p[:, 1], pos_p[:, 2]
    n2 = x * x + y * y + zc * zc
    bf = batch_p.astype(jnp.float32)
    one = jnp.ones_like(n2)
    feati = jnp.stack([x, y, zc, n2, one, bf * bf, -2.0 * bf, one], axis=1)
    featj = jnp.stack([-2.0 * x, -2.0 * y, -2.0 * zc, one, n2, one, bf,
                       bf * bf], axis=1)

    npad_j = npad + TJ
    featj = jnp.pad(featj, ((0, TJ), (0, 0)))
    featj = featj.at[npad:, 6].set(jnp.float32(2 ** 30))
    featj = featj.at[npad:, 7].set(jnp.float32(2 ** 60))

    r2 = lambda a: a.reshape(1, -1)

    full = lambda shape: pl.BlockSpec(shape, lambda t: (0,) * len(shape))
    rowblk = lambda w: pl.BlockSpec((TI, w), lambda t: (t, 0))
    smem = pl.BlockSpec(memory_space=pltpu.SMEM)

    v = pl.pallas_call(
        _embed_kernel,
        grid=(num_tiles,),
        in_specs=[rowblk(4), full(emb1.shape), full(emb2.shape),
                  full(emb3.shape), full(emb4.shape), full(lin1_W.shape),
                  full((1, H))],
        out_specs=rowblk(H),
        out_shape=jax.ShapeDtypeStruct((npad, H), jnp.float32),
        interpret=interpret,
    )(z_p, emb1, emb2, emb3, emb4, lin1_W, r2(lin1_b))

    rowblk_l = lambda w: pl.BlockSpec((TIL, w), lambda t: (t, 0))
    layer_call = pl.pallas_call(
        _layer_kernel,
        grid=(num_itiles,),
        in_specs=[smem, smem, rowblk_l(8), full((npad_j, 8)),
                  rowblk_l(H), full((npad_j, H)),
                  full((H, G)), full((1, H)), full((H, H)), full((1, H)),
                  full((H, H)), full((1, H)), full((H, H)), full((1, H))],
        out_specs=rowblk_l(H),
        out_shape=jax.ShapeDtypeStruct((npad, H), jnp.float32),
        interpret=interpret,
    )

    vl_call = pl.pallas_call(
        _vl_kernel,
        grid=(num_tiles,),
        in_specs=[rowblk(H), full((H, H))],
        out_specs=rowblk(H),
        out_shape=jax.ShapeDtypeStruct((npad, H), jnp.float32),
        interpret=interpret,
    )

    xs = []
    for l in range(3):
        vl_j = jnp.pad(vl_call(v, e_lin_W[l]), ((0, TJ), (0, 0)))
        v = layer_call(jstart8, jnum, feati, featj, v, vl_j,
                       e_mlp1_W[l], r2(e_mlp1_b[l]), e_mlp2_W[l],
                       r2(e_mlp2_b[l]), v_lin1_W[l], r2(v_lin1_b[l]),
                       v_lin2_W[l], r2(v_lin2_b[l]))
        xs.append(v)

    out = pl.pallas_call(
        functools.partial(_final_kernel, num_graphs=num_graphs),
        grid=(num_tiles,),
        in_specs=[rowblk(H), rowblk(H), rowblk(H), full((1, npad)),
                  full(u_lin1_W.shape), full((1, H)), full(u_lin2_W.shape),
                  full((1, H)), smem],
        out_specs=pl.BlockSpec((num_graphs, H), lambda t: (0, 0)),
        out_shape=jax.ShapeDtypeStruct((num_graphs, H), jnp.float32),
        scratch_shapes=[pltpu.VMEM((num_graphs, H), jnp.float32)],
        interpret=interpret,
    )(xs[0], xs[1], xs[2], batchT, u_lin1_W, r2(u_lin1_b), u_lin2_W,
      r2(u_lin2_b), jnp.array([num_tiles], jnp.int32))
    return out


def kernel(z, pos, batch, emb1, emb2, emb3, emb4, lin1_W, lin1_b, e_mlp1_W,
           e_mlp1_b, e_mlp2_W, e_mlp2_b, e_lin_W, v_lin1_W, v_lin1_b,
           v_lin2_W, v_lin2_b, u_lin1_W, u_lin1_b, u_lin2_W, u_lin2_b):
    return _forward(z, pos, batch, emb1, emb2, emb3, emb4, lin1_W, lin1_b,
                    e_mlp1_W, e_mlp1_b, e_mlp2_W, e_mlp2_b, e_lin_W, v_lin1_W,
                    v_lin1_b, v_lin2_W, v_lin2_b, u_lin1_W, u_lin1_b, u_lin2_W,
                    u_lin2_b, NUM_GRAPHS)


# speedup vs baseline: 1.0613x; 1.0613x over previous
import functools

import jax
import jax.numpy as jnp
from jax import lax
from jax.experimental import pallas as pl
from jax.experimental.pallas import tpu as pltpu

CUTOFF = 5.0
G = 50
H = 128
NUM_GRAPHS = 256
TI = 128
TIL = 64
TJ = 128


def _embed_kernel(z_ref, emb1_ref, emb2_ref, emb3_ref, emb4_ref, W_ref, b_ref,
                  v_ref):
    z = z_ref[...]

    def onehot(col, k):
        ids = z[:, col][:, None]
        return (ids == lax.broadcasted_iota(jnp.int32, (TI, k), 1)).astype(
            jnp.float32)

    e1 = jnp.dot(onehot(0, 100), emb1_ref[...],
                 preferred_element_type=jnp.float32)
    e2 = jnp.dot(onehot(1, 10), emb2_ref[...],
                 preferred_element_type=jnp.float32)
    e3 = jnp.dot(onehot(2, 10), emb3_ref[...],
                 preferred_element_type=jnp.float32)
    e4 = jnp.dot(onehot(3, 10), emb4_ref[...],
                 preferred_element_type=jnp.float32)
    W = W_ref[...]
    v = (jnp.dot(e1, W[:, 0:H].T, preferred_element_type=jnp.float32)
         + jnp.dot(e2, W[:, H:2 * H].T, preferred_element_type=jnp.float32)
         + jnp.dot(e3, W[:, 2 * H:3 * H].T, preferred_element_type=jnp.float32)
         + jnp.dot(e4, W[:, 3 * H:4 * H].T, preferred_element_type=jnp.float32)
         + b_ref[...])
    v_ref[...] = v


def _layer_kernel(jstart8_ref, jnum_ref, feati_ref, featj_ref, v_ref, vl_ref,
                  W1_ref, b1_ref, W2_ref, b2_ref, v1W_ref, v1b_ref, v2W_ref,
                  v2b_ref, vout_ref):
    t = pl.program_id(0)
    i0 = t * TIL
    iidx = i0 + lax.broadcasted_iota(jnp.int32, (TIL, 1), 0)

    step = CUTOFF / (G - 1)
    coeff = -0.5 / (step * step)
    offset = step * lax.broadcasted_iota(jnp.int32, (1, 1, G), 2).astype(
        jnp.float32)
    sg = -2.0 * coeff * offset
    tg = coeff * offset * offset

    W1t = W1_ref[...].T
    W2t = W2_ref[...].T
    b1 = b1_ref[...]
    b2 = b2_ref[...]
    jbase = jstart8_ref[t] * 8
    feati = feati_ref[...]
    fi_pos = feati[:, 0:5]
    fi_b = feati[:, 5:8]

    def body(jt, acc):
        j0 = jbase + jt * TJ
        featj = featj_ref[pl.ds(j0, TJ), :]
        vlj = vl_ref[pl.ds(j0, TJ), :]

        cn = (((1,), (1,)), ((), ()))
        d2 = lax.dot_general(fi_pos, featj[:, 0:5], cn,
                             precision=lax.Precision.HIGHEST,
                             preferred_element_type=jnp.float32)
        db2 = lax.dot_general(fi_b, featj[:, 5:8], cn,
                              precision=lax.Precision.HIGHEST,
                              preferred_element_type=jnp.float32)
        d2 = jnp.maximum(d2, 0.0)
        d = jnp.sqrt(d2)
        jidx = j0 + lax.broadcasted_iota(jnp.int32, (1, TJ), 1)
        mask = (d < CUTOFF) & (db2 < 0.5) & (iidx != jidx)
        C = 0.5 * (jnp.cos(d * (jnp.pi / CUTOFF)) + 1.0)
        scal = jnp.where(mask, C, 0.0)

        q = coeff * d2
        de = jnp.exp(q[:, :, None] + d[:, :, None] * sg + tg)
        de_f = de.reshape(TIL * TJ, G)
        h = jnp.maximum(
            jnp.dot(de_f, W1t, preferred_element_type=jnp.float32) + b1, 0.0)
        wg = jnp.dot(h, W2t, preferred_element_type=jnp.float32) + b2
        wg3 = wg.reshape(TIL, TJ, H)
        contrib = wg3 * (scal[:, :, None] * vlj[None, :, :])
        return acc + jnp.sum(contrib, axis=1)

    acc = lax.fori_loop(0, jnum_ref[t], body,
                        jnp.zeros((TIL, H), jnp.float32))

    vi = v_ref[...]
    o = jnp.maximum(
        jnp.dot(acc, v1W_ref[...].T, preferred_element_type=jnp.float32)
        + v1b_ref[...], 0.0)
    o = jnp.dot(o, v2W_ref[...].T, preferred_element_type=jnp.float32) \
        + v2b_ref[...]
    vout_ref[...] = vi + o


def _vl_kernel(v_ref, W_ref, out_ref):
    out_ref[...] = jnp.dot(v_ref[...], W_ref[...].T,
                           preferred_element_type=jnp.float32)


def _final_kernel(x1_ref, x2_ref, x3_ref, batchT_ref, W1_ref, b1_ref, W2_ref,
                  b2_ref, num_tiles_ref, out_ref, acc_ref, *, num_graphs):
    t = pl.program_id(0)

    @pl.when(t == 0)
    def _():
        acc_ref[...] = jnp.zeros_like(acc_ref)

    W1 = W1_ref[...]
    u = (jnp.dot(x1_ref[...], W1[:, 0:H].T, preferred_element_type=jnp.float32)
         + jnp.dot(x2_ref[...], W1[:, H:2 * H].T,
                   preferred_element_type=jnp.float32)
         + jnp.dot(x3_ref[...], W1[:, 2 * H:3 * H].T,
                   preferred_element_type=jnp.float32)
         + b1_ref[...])
    u = jnp.maximum(u, 0.0)

    bj = batchT_ref[:, pl.ds(t * TI, TI)]
    ohT = (lax.broadcasted_iota(jnp.int32, (num_graphs, TI), 0) == bj).astype(
        jnp.float32)
    acc_ref[...] += jnp.dot(ohT, u, preferred_element_type=jnp.float32)

    @pl.when(t == num_tiles_ref[0] - 1)
    def _():
        out_ref[...] = jnp.maximum(
            jnp.dot(acc_ref[...], W2_ref[...].T,
                    preferred_element_type=jnp.float32) + b2_ref[...], 0.0)


def _forward(z, pos, batch, emb1, emb2, emb3, emb4, lin1_W, lin1_b, e_mlp1_W,
             e_mlp1_b, e_mlp2_W, e_mlp2_b, e_lin_W, v_lin1_W, v_lin1_b,
             v_lin2_W, v_lin2_b, u_lin1_W, u_lin1_b, u_lin2_W, u_lin2_b,
             num_graphs, interpret=False):
    n = pos.shape[0]
    num_tiles = (n + TI - 1) // TI
    npad = num_tiles * TI
    pad = npad - n

    batch = batch.astype(jnp.int32)
    z = z.astype(jnp.int32)
    pos_p = jnp.pad(pos, ((0, pad), (0, 0)))
    batch_p = jnp.pad(batch, (0, pad), constant_values=jnp.int32(2 ** 30))
    z_p = jnp.pad(z, ((0, pad), (0, 0)))
    batchT = batch_p[None, :]

    num_itiles = npad // TIL
    tstarts = jnp.arange(num_itiles, dtype=jnp.int32) * TIL
    row_lo = jnp.clip(tstarts, 0, n - 1)
    row_hi = jnp.clip(tstarts + TIL - 1, 0, n - 1)
    blo = batch[row_lo]
    bhi = batch[row_hi]
    jlo = jnp.searchsorted(batch, blo, side="left").astype(jnp.int32)
    jhi = jnp.searchsorted(batch, bhi, side="right").astype(jnp.int32)
    jstart8 = jlo // 8
    jnum = (jhi - jstart8 * 8 + TJ - 1) // TJ

    x, y, zc = pos_p[:, 0], pos_p[:, 1], pos_p[:, 2]
    n2 = x * x + y * y + zc * zc
    bf = batch_p.astype(jnp.float32)
    one = jnp.ones_like(n2)
    feati = jnp.stack([x, y, zc, n2, one, bf * bf, -2.0 * bf, one], axis=1)
    featj = jnp.stack([-2.0 * x, -2.0 * y, -2.0 * zc, one, n2, one, bf,
                       bf * bf], axis=1)

    npad_j = npad + TJ
    featj = jnp.pad(featj, ((0, TJ), (0, 0)))
    featj = featj.at[npad:, 6].set(jnp.float32(2 ** 30))
    featj = featj.at[npad:, 7].set(jnp.float32(2 ** 60))

    r2 = lambda a: a.reshape(1, -1)

    full = lambda shape: pl.BlockSpec(shape, lambda t: (0,) * len(shape))
    rowblk = lambda w: pl.BlockSpec((TI, w), lambda t: (t, 0))
    smem = pl.BlockSpec(memory_space=pltpu.SMEM)

    v = pl.pallas_call(
        _embed_kernel,
        grid=(num_tiles,),
        in_specs=[rowblk(4), full(emb1.shape), full(emb2.shape),
                  full(emb3.shape), full(emb4.shape), full(lin1_W.shape),
                  full((1, H))],
        out_specs=rowblk(H),
        out_shape=jax.ShapeDtypeStruct((npad, H), jnp.float32),
        interpret=interpret,
    )(z_p, emb1, emb2, emb3, emb4, lin1_W, r2(lin1_b))

    rowblk_l = lambda w: pl.BlockSpec((TIL, w), lambda t: (t, 0))
    layer_call = pl.pallas_call(
        _layer_kernel,
        grid=(num_itiles,),
        in_specs=[smem, smem, rowblk_l(8), full((npad_j, 8)),
                  rowblk_l(H), full((npad_j, H)),
                  full((H, G)), full((1, H)), full((H, H)), full((1, H)),
                  full((H, H)), full((1, H)), full((H, H)), full((1, H))],
        out_specs=rowblk_l(H),
        out_shape=jax.ShapeDtypeStruct((npad, H), jnp.float32),
        interpret=interpret,
    )

    vl_call = pl.pallas_call(
        _vl_kernel,
        grid=(num_tiles,),
        in_specs=[rowblk(H), full((H, H))],
        out_specs=rowblk(H),
        out_shape=jax.ShapeDtypeStruct((npad, H), jnp.float32),
        interpret=interpret,
    )

    xs = []
    for l in range(3):
        vl_j = jnp.pad(vl_call(v, e_lin_W[l]), ((0, TJ), (0, 0)))
        v = layer_call(jstart8, jnum, feati, featj, v, vl_j,
                       e_mlp1_W[l], r2(e_mlp1_b[l]), e_mlp2_W[l],
                       r2(e_mlp2_b[l]), v_lin1_W[l], r2(v_lin1_b[l]),
                       v_lin2_W[l], r2(v_lin2_b[l]))
        xs.append(v)

    out = pl.pallas_call(
        functools.partial(_final_kernel, num_graphs=num_graphs),
        grid=(num_tiles,),
        in_specs=[rowblk(H), rowblk(H), rowblk(H), full((1, npad)),
                  full(u_lin1_W.shape), full((1, H)), full(u_lin2_W.shape),
                  full((1, H)), smem],
        out_specs=pl.BlockSpec((num_graphs, H), lambda t: (0, 0)),
        out_shape=jax.ShapeDtypeStruct((num_graphs, H), jnp.float32),
        scratch_shapes=[pltpu.VMEM((num_graphs, H), jnp.float32)],
        interpret=interpret,
    )(xs[0], xs[1], xs[2], batchT, u_lin1_W, r2(u_lin1_b), u_lin2_W,
      r2(u_lin2_b), jnp.array([num_tiles], jnp.int32))
    return out


def kernel(z, pos, batch, emb1, emb2, emb3, emb4, lin1_W, lin1_b, e_mlp1_W,
           e_mlp1_b, e_mlp2_W, e_mlp2_b, e_lin_W, v_lin1_W, v_lin1_b,
           v_lin2_W, v_lin2_b, u_lin1_W, u_lin1_b, u_lin2_W, u_lin2_b):
    return _forward(z, pos, batch, emb1, emb2, emb3, emb4, lin1_W, lin1_b,
                    e_mlp1_W, e_mlp1_b, e_mlp2_W, e_mlp2_b, e_lin_W, v_lin1_W,
                    v_lin1_b, v_lin2_W, v_lin2_b, u_lin1_W, u_lin1_b, u_lin2_W,
                    u_lin2_b, NUM_GRAPHS)


# speedup vs baseline: 1.3510x; 1.2729x over previous
import functools

import jax
import jax.numpy as jnp
from jax import lax
from jax.experimental import pallas as pl
from jax.experimental.pallas import tpu as pltpu

CUTOFF = 5.0
G = 50
H = 128
NUM_GRAPHS = 256
TI = 128
TIL = 64
TJ = 128


def _embed_kernel(z_ref, emb1_ref, emb2_ref, emb3_ref, emb4_ref, W_ref, b_ref,
                  v_ref):
    z = z_ref[...]

    def onehot(col, k):
        ids = z[:, col][:, None]
        return (ids == lax.broadcasted_iota(jnp.int32, (TI, k), 1)).astype(
            jnp.float32)

    e1 = jnp.dot(onehot(0, 100), emb1_ref[...],
                 preferred_element_type=jnp.float32)
    e2 = jnp.dot(onehot(1, 10), emb2_ref[...],
                 preferred_element_type=jnp.float32)
    e3 = jnp.dot(onehot(2, 10), emb3_ref[...],
                 preferred_element_type=jnp.float32)
    e4 = jnp.dot(onehot(3, 10), emb4_ref[...],
                 preferred_element_type=jnp.float32)
    W = W_ref[...]
    v = (jnp.dot(e1, W[:, 0:H].T, preferred_element_type=jnp.float32)
         + jnp.dot(e2, W[:, H:2 * H].T, preferred_element_type=jnp.float32)
         + jnp.dot(e3, W[:, 2 * H:3 * H].T, preferred_element_type=jnp.float32)
         + jnp.dot(e4, W[:, 3 * H:4 * H].T, preferred_element_type=jnp.float32)
         + b_ref[...])
    v_ref[...] = v


def _layer_kernel(jstart8_ref, jnum_ref, feati_ref, featj_ref, v_ref, vl_ref,
                  W1_ref, b1_ref, W2_ref, b2_ref, v1W_ref, v1b_ref, v2W_ref,
                  v2b_ref, vout_ref):
    t = pl.program_id(0)
    i0 = t * TIL
    iidx = i0 + lax.broadcasted_iota(jnp.int32, (TIL, 1), 0)

    step = CUTOFF / (G - 1)
    coeff = -0.5 / (step * step)
    offset = step * lax.broadcasted_iota(jnp.int32, (1, 1, G), 2).astype(
        jnp.float32)
    sg = -2.0 * coeff * offset
    tg = coeff * offset * offset

    W1t = W1_ref[...].T
    W2t = W2_ref[...].T
    b1 = b1_ref[...]
    b2 = b2_ref[...]
    jbase = jstart8_ref[t] * 8
    feati = feati_ref[...]
    fi_pos = feati[:, 0:5]
    fi_b = feati[:, 5:8]

    def body(jt, acc):
        j0 = jbase + jt * TJ
        featj = featj_ref[pl.ds(j0, TJ), :]
        vlj = vl_ref[pl.ds(j0, TJ), :]

        cn = (((1,), (1,)), ((), ()))
        d2 = lax.dot_general(fi_pos, featj[:, 0:5], cn,
                             precision=lax.Precision.HIGHEST,
                             preferred_element_type=jnp.float32)
        db2 = lax.dot_general(fi_b, featj[:, 5:8], cn,
                              precision=lax.Precision.HIGHEST,
                              preferred_element_type=jnp.float32)
        d2 = jnp.maximum(d2, 0.0)
        d = jnp.sqrt(d2)
        jidx = j0 + lax.broadcasted_iota(jnp.int32, (1, TJ), 1)
        mask = (d < CUTOFF) & (db2 < 0.5) & (iidx != jidx)
        C = 0.5 * (jnp.cos(d * (jnp.pi / CUTOFF)) + 1.0)
        scal = jnp.where(mask, C, 0.0)

        de = jnp.exp(coeff * (d[:, :, None] - offset) ** 2)
        de_f = de.reshape(TIL * TJ, G)
        h = jnp.maximum(
            jnp.dot(de_f, W1t, preferred_element_type=jnp.float32) + b1, 0.0)
        wg = jnp.dot(h, W2t, preferred_element_type=jnp.float32) + b2
        wg3 = wg.reshape(TIL, TJ, H)
        contrib = wg3 * (scal[:, :, None] * vlj[None, :, :])
        return acc + jnp.sum(contrib, axis=1)

    acc = lax.fori_loop(0, jnum_ref[t], body,
                        jnp.zeros((TIL, H), jnp.float32))

    vi = v_ref[...]
    o = jnp.maximum(
        jnp.dot(acc, v1W_ref[...].T, preferred_element_type=jnp.float32)
        + v1b_ref[...], 0.0)
    o = jnp.dot(o, v2W_ref[...].T, preferred_element_type=jnp.float32) \
        + v2b_ref[...]
    vout_ref[...] = vi + o


def _vl_kernel(v_ref, W_ref, out_ref):
    out_ref[...] = jnp.dot(v_ref[...], W_ref[...].T,
                           preferred_element_type=jnp.float32)


def _final_kernel(x1_ref, x2_ref, x3_ref, batchT_ref, W1_ref, b1_ref, W2_ref,
                  b2_ref, num_tiles_ref, out_ref, acc_ref, *, num_graphs):
    t = pl.program_id(0)

    @pl.when(t == 0)
    def _():
        acc_ref[...] = jnp.zeros_like(acc_ref)

    W1 = W1_ref[...]
    u = (jnp.dot(x1_ref[...], W1[:, 0:H].T, preferred_element_type=jnp.float32)
         + jnp.dot(x2_ref[...], W1[:, H:2 * H].T,
                   preferred_element_type=jnp.float32)
         + jnp.dot(x3_ref[...], W1[:, 2 * H:3 * H].T,
                   preferred_element_type=jnp.float32)
         + b1_ref[...])
    u = jnp.maximum(u, 0.0)

    bj = batchT_ref[:, pl.ds(t * TI, TI)]
    ohT = (lax.broadcasted_iota(jnp.int32, (num_graphs, TI), 0) == bj).astype(
        jnp.float32)
    acc_ref[...] += jnp.dot(ohT, u, preferred_element_type=jnp.float32)

    @pl.when(t == num_tiles_ref[0] - 1)
    def _():
        out_ref[...] = jnp.maximum(
            jnp.dot(acc_ref[...], W2_ref[...].T,
                    preferred_element_type=jnp.float32) + b2_ref[...], 0.0)


def _forward(z, pos, batch, emb1, emb2, emb3, emb4, lin1_W, lin1_b, e_mlp1_W,
             e_mlp1_b, e_mlp2_W, e_mlp2_b, e_lin_W, v_lin1_W, v_lin1_b,
             v_lin2_W, v_lin2_b, u_lin1_W, u_lin1_b, u_lin2_W, u_lin2_b,
             num_graphs, interpret=False):
    n = pos.shape[0]
    num_tiles = (n + TI - 1) // TI
    npad = num_tiles * TI
    pad = npad - n

    batch = batch.astype(jnp.int32)
    z = z.astype(jnp.int32)
    pos_p = jnp.pad(pos, ((0, pad), (0, 0)))
    batch_p = jnp.pad(batch, (0, pad), constant_values=jnp.int32(2 ** 30))
    z_p = jnp.pad(z, ((0, pad), (0, 0)))
    batchT = batch_p[None, :]

    num_itiles = npad // TIL
    tstarts = jnp.arange(num_itiles, dtype=jnp.int32) * TIL
    row_lo = jnp.clip(tstarts, 0, n - 1)
    row_hi = jnp.clip(tstarts + TIL - 1, 0, n - 1)
    blo = batch[row_lo]
    bhi = batch[row_hi]
    jlo = jnp.searchsorted(batch, blo, side="left").astype(jnp.int32)
    jhi = jnp.searchsorted(batch, bhi, side="right").astype(jnp.int32)
    jstart8 = jlo // 8
    jnum = (jhi - jstart8 * 8 + TJ - 1) // TJ

    x, y, zc = pos_p[:, 0], pos_p[:, 1], pos_p[:, 2]
    n2 = x * x + y * y + zc * zc
    bf = batch_p.astype(jnp.float32)
    one = jnp.ones_like(n2)
    feati = jnp.stack([x, y, zc, n2, one, bf * bf, -2.0 * bf, one], axis=1)
    featj = jnp.stack([-2.0 * x, -2.0 * y, -2.0 * zc, one, n2, one, bf,
                       bf * bf], axis=1)

    npad_j = npad + TJ
    featj = jnp.pad(featj, ((0, TJ), (0, 0)))
    featj = featj.at[npad:, 6].set(jnp.float32(2 ** 30))
    featj = featj.at[npad:, 7].set(jnp.float32(2 ** 60))

    r2 = lambda a: a.reshape(1, -1)

    full = lambda shape: pl.BlockSpec(shape, lambda t: (0,) * len(shape))
    rowblk = lambda w: pl.BlockSpec((TI, w), lambda t: (t, 0))
    smem = pl.BlockSpec(memory_space=pltpu.SMEM)

    v = pl.pallas_call(
        _embed_kernel,
        grid=(num_tiles,),
        in_specs=[rowblk(4), full(emb1.shape), full(emb2.shape),
                  full(emb3.shape), full(emb4.shape), full(lin1_W.shape),
                  full((1, H))],
        out_specs=rowblk(H),
        out_shape=jax.ShapeDtypeStruct((npad, H), jnp.float32),
        interpret=interpret,
    )(z_p, emb1, emb2, emb3, emb4, lin1_W, r2(lin1_b))

    rowblk_l = lambda w: pl.BlockSpec((TIL, w), lambda t: (t, 0))
    layer_call = pl.pallas_call(
        _layer_kernel,
        grid=(num_itiles,),
        in_specs=[smem, smem, rowblk_l(8), full((npad_j, 8)),
                  rowblk_l(H), full((npad_j, H)),
                  full((H, G)), full((1, H)), full((H, H)), full((1, H)),
                  full((H, H)), full((1, H)), full((H, H)), full((1, H))],
        out_specs=rowblk_l(H),
        out_shape=jax.ShapeDtypeStruct((npad, H), jnp.float32),
        interpret=interpret,
    )

    vl_call = pl.pallas_call(
        _vl_kernel,
        grid=(num_tiles,),
        in_specs=[rowblk(H), full((H, H))],
        out_specs=rowblk(H),
        out_shape=jax.ShapeDtypeStruct((npad, H), jnp.float32),
        interpret=interpret,
    )

    xs = []
    for l in range(3):
        vl_j = jnp.pad(vl_call(v, e_lin_W[l]), ((0, TJ), (0, 0)))
        v = layer_call(jstart8, jnum, feati, featj, v, vl_j,
                       e_mlp1_W[l], r2(e_mlp1_b[l]), e_mlp2_W[l],
                       r2(e_mlp2_b[l]), v_lin1_W[l], r2(v_lin1_b[l]),
                       v_lin2_W[l], r2(v_lin2_b[l]))
        xs.append(v)

    out = pl.pallas_call(
        functools.partial(_final_kernel, num_graphs=num_graphs),
        grid=(num_tiles,),
        in_specs=[rowblk(H), rowblk(H), rowblk(H), full((1, npad)),
                  full(u_lin1_W.shape), full((1, H)), full(u_lin2_W.shape),
                  full((1, H)), smem],
        out_specs=pl.BlockSpec((num_graphs, H), lambda t: (0, 0)),
        out_shape=jax.ShapeDtypeStruct((num_graphs, H), jnp.float32),
        scratch_shapes=[pltpu.VMEM((num_graphs, H), jnp.float32)],
        interpret=interpret,
    )(xs[0], xs[1], xs[2], batchT, u_lin1_W, r2(u_lin1_b), u_lin2_W,
      r2(u_lin2_b), jnp.array([num_tiles], jnp.int32))
    return out


def kernel(z, pos, batch, emb1, emb2, emb3, emb4, lin1_W, lin1_b, e_mlp1_W,
           e_mlp1_b, e_mlp2_W, e_mlp2_b, e_lin_W, v_lin1_W, v_lin1_b,
           v_lin2_W, v_lin2_b, u_lin1_W, u_lin1_b, u_lin2_W, u_lin2_b):
    return _forward(z, pos, batch, emb1, emb2, emb3, emb4, lin1_W, lin1_b,
                    e_mlp1_W, e_mlp1_b, e_mlp2_W, e_mlp2_b, e_lin_W, v_lin1_W,
                    v_lin1_b, v_lin2_W, v_lin2_b, u_lin1_W, u_lin1_b, u_lin2_W,
                    u_lin2_b, NUM_GRAPHS)


# speedup vs baseline: 1.4706x; 1.0886x over previous
import functools

import jax
import jax.numpy as jnp
from jax import lax
from jax.experimental import pallas as pl
from jax.experimental.pallas import tpu as pltpu

CUTOFF = 5.0
G = 50
H = 128
NUM_GRAPHS = 256
TI = 128
TIL = 64
TJ = 128


def _embed_kernel(z_ref, emb1_ref, emb2_ref, emb3_ref, emb4_ref, W_ref, b_ref,
                  v_ref):
    z = z_ref[...]

    def onehot(col, k):
        ids = z[:, col][:, None]
        return (ids == lax.broadcasted_iota(jnp.int32, (TI, k), 1)).astype(
            jnp.float32)

    e1 = jnp.dot(onehot(0, 100), emb1_ref[...],
                 preferred_element_type=jnp.float32)
    e2 = jnp.dot(onehot(1, 10), emb2_ref[...],
                 preferred_element_type=jnp.float32)
    e3 = jnp.dot(onehot(2, 10), emb3_ref[...],
                 preferred_element_type=jnp.float32)
    e4 = jnp.dot(onehot(3, 10), emb4_ref[...],
                 preferred_element_type=jnp.float32)
    W = W_ref[...]
    v = (jnp.dot(e1, W[:, 0:H].T, preferred_element_type=jnp.float32)
         + jnp.dot(e2, W[:, H:2 * H].T, preferred_element_type=jnp.float32)
         + jnp.dot(e3, W[:, 2 * H:3 * H].T, preferred_element_type=jnp.float32)
         + jnp.dot(e4, W[:, 3 * H:4 * H].T, preferred_element_type=jnp.float32)
         + b_ref[...])
    v_ref[...] = v


def _layer_kernel(jstart8_ref, jnum_ref, feati_ref, featj_ref, v_ref,
                  elW_ref, W1_ref, b1_ref, W2_ref, b2_ref, v1W_ref, v1b_ref,
                  v2W_ref, v2b_ref, vout_ref):
    t = pl.program_id(0)
    i0 = t * TIL
    iidx = i0 + lax.broadcasted_iota(jnp.int32, (TIL, 1), 0)

    step = CUTOFF / (G - 1)
    coeff = -0.5 / (step * step)
    offset = step * lax.broadcasted_iota(jnp.int32, (1, 1, G), 2).astype(
        jnp.float32)
    W1t = W1_ref[...].T
    W2t = W2_ref[...].T
    b1 = b1_ref[...]
    b2 = b2_ref[...]
    elWt = elW_ref[...].T
    jbase = jstart8_ref[t] * 8
    feati = feati_ref[...]
    fi_pos = feati[:, 0:5]
    fi_b = feati[:, 5:8]

    def body(jt, acc):
        j0 = jbase + jt * TJ
        featj = featj_ref[pl.ds(j0, TJ), :]
        vj = v_ref[pl.ds(j0, TJ), :]
        vlj = jnp.dot(vj, elWt, preferred_element_type=jnp.float32)

        cn = (((1,), (1,)), ((), ()))
        d2 = lax.dot_general(fi_pos, featj[:, 0:5], cn,
                             precision=lax.Precision.HIGHEST,
                             preferred_element_type=jnp.float32)
        db2 = lax.dot_general(fi_b, featj[:, 5:8], cn,
                              precision=lax.Precision.HIGHEST,
                              preferred_element_type=jnp.float32)
        d2 = jnp.maximum(d2, 0.0)
        d = jnp.sqrt(d2)
        jidx = j0 + lax.broadcasted_iota(jnp.int32, (1, TJ), 1)
        mask = (d < CUTOFF) & (db2 < 0.5) & (iidx != jidx)
        C = 0.5 * (jnp.cos(d * (jnp.pi / CUTOFF)) + 1.0)
        scal = jnp.where(mask, C, 0.0)

        de = jnp.exp(coeff * (d[:, :, None] - offset) ** 2)
        de_f = de.reshape(TIL * TJ, G)
        h = jnp.maximum(
            jnp.dot(de_f, W1t, preferred_element_type=jnp.float32) + b1, 0.0)
        wg = jnp.dot(h, W2t, preferred_element_type=jnp.float32) + b2
        wg3 = wg.reshape(TIL, TJ, H)
        contrib = wg3 * (scal[:, :, None] * vlj[None, :, :])
        return acc + jnp.sum(contrib, axis=1)

    acc = lax.fori_loop(0, jnum_ref[t], body,
                        jnp.zeros((TIL, H), jnp.float32))

    vi = v_ref[pl.ds(i0, TIL), :]
    o = jnp.maximum(
        jnp.dot(acc, v1W_ref[...].T, preferred_element_type=jnp.float32)
        + v1b_ref[...], 0.0)
    o = jnp.dot(o, v2W_ref[...].T, preferred_element_type=jnp.float32) \
        + v2b_ref[...]
    vout_ref[...] = vi + o


def _final_kernel(x1_ref, x2_ref, x3_ref, batchT_ref, W1_ref, b1_ref, W2_ref,
                  b2_ref, num_tiles_ref, out_ref, acc_ref, *, num_graphs):
    t = pl.program_id(0)

    @pl.when(t == 0)
    def _():
        acc_ref[...] = jnp.zeros_like(acc_ref)

    W1 = W1_ref[...]
    u = (jnp.dot(x1_ref[...], W1[:, 0:H].T, preferred_element_type=jnp.float32)
         + jnp.dot(x2_ref[...], W1[:, H:2 * H].T,
                   preferred_element_type=jnp.float32)
         + jnp.dot(x3_ref[...], W1[:, 2 * H:3 * H].T,
                   preferred_element_type=jnp.float32)
         + b1_ref[...])
    u = jnp.maximum(u, 0.0)

    bj = batchT_ref[:, pl.ds(t * TI, TI)]
    ohT = (lax.broadcasted_iota(jnp.int32, (num_graphs, TI), 0) == bj).astype(
        jnp.float32)
    acc_ref[...] += jnp.dot(ohT, u, preferred_element_type=jnp.float32)

    @pl.when(t == num_tiles_ref[0] - 1)
    def _():
        out_ref[...] = jnp.maximum(
            jnp.dot(acc_ref[...], W2_ref[...].T,
                    preferred_element_type=jnp.float32) + b2_ref[...], 0.0)


def _forward(z, pos, batch, emb1, emb2, emb3, emb4, lin1_W, lin1_b, e_mlp1_W,
             e_mlp1_b, e_mlp2_W, e_mlp2_b, e_lin_W, v_lin1_W, v_lin1_b,
             v_lin2_W, v_lin2_b, u_lin1_W, u_lin1_b, u_lin2_W, u_lin2_b,
             num_graphs, interpret=False):
    n = pos.shape[0]
    num_tiles = (n + TI - 1) // TI
    npad = num_tiles * TI
    pad = npad - n

    batch = batch.astype(jnp.int32)
    z = z.astype(jnp.int32)
    pos_p = jnp.pad(pos, ((0, pad), (0, 0)))
    batch_p = jnp.pad(batch, (0, pad), constant_values=jnp.int32(2 ** 30))
    z_p = jnp.pad(z, ((0, pad), (0, 0)))
    batchT = batch_p[None, :]

    num_itiles = npad // TIL
    tstarts = jnp.arange(num_itiles, dtype=jnp.int32) * TIL
    row_lo = jnp.clip(tstarts, 0, n - 1)
    row_hi = jnp.clip(tstarts + TIL - 1, 0, n - 1)
    blo = batch[row_lo]
    bhi = batch[row_hi]
    jlo = jnp.searchsorted(batch, blo, side="left").astype(jnp.int32)
    jhi = jnp.searchsorted(batch, bhi, side="right").astype(jnp.int32)
    jstart8 = jlo // 8
    jnum = (jhi - jstart8 * 8 + TJ - 1) // TJ

    x, y, zc = pos_p[:, 0], pos_p[:, 1], pos_p[:, 2]
    n2 = x * x + y * y + zc * zc
    bf = batch_p.astype(jnp.float32)
    one = jnp.ones_like(n2)
    feati = jnp.stack([x, y, zc, n2, one, bf * bf, -2.0 * bf, one], axis=1)
    featj = jnp.stack([-2.0 * x, -2.0 * y, -2.0 * zc, one, n2, one, bf,
                       bf * bf], axis=1)

    npad_j = npad + TJ
    featj = jnp.pad(featj, ((0, TJ), (0, 0)))
    featj = featj.at[npad:, 6].set(jnp.float32(2 ** 30))
    featj = featj.at[npad:, 7].set(jnp.float32(2 ** 60))

    r2 = lambda a: a.reshape(1, -1)

    full = lambda shape: pl.BlockSpec(shape, lambda t: (0,) * len(shape))
    rowblk = lambda w: pl.BlockSpec((TI, w), lambda t: (t, 0))
    smem = pl.BlockSpec(memory_space=pltpu.SMEM)

    v = pl.pallas_call(
        _embed_kernel,
        grid=(num_tiles,),
        in_specs=[rowblk(4), full(emb1.shape), full(emb2.shape),
                  full(emb3.shape), full(emb4.shape), full(lin1_W.shape),
                  full((1, H))],
        out_specs=rowblk(H),
        out_shape=jax.ShapeDtypeStruct((npad, H), jnp.float32),
        interpret=interpret,
    )(z_p, emb1, emb2, emb3, emb4, lin1_W, r2(lin1_b))

    rowblk_l = lambda w: pl.BlockSpec((TIL, w), lambda t: (t, 0))
    layer_call = pl.pallas_call(
        _layer_kernel,
        grid=(num_itiles,),
        in_specs=[smem, smem, rowblk_l(8), full((npad_j, 8)),
                  full((npad_j, H)), full((H, H)),
                  full((H, G)), full((1, H)), full((H, H)), full((1, H)),
                  full((H, H)), full((1, H)), full((H, H)), full((1, H))],
        out_specs=rowblk_l(H),
        out_shape=jax.ShapeDtypeStruct((npad, H), jnp.float32),
        interpret=interpret,
    )

    xs = []
    for l in range(3):
        v_j = jnp.pad(v, ((0, TJ), (0, 0)))
        v = layer_call(jstart8, jnum, feati, featj, v_j, e_lin_W[l],
                       e_mlp1_W[l], r2(e_mlp1_b[l]), e_mlp2_W[l],
                       r2(e_mlp2_b[l]), v_lin1_W[l], r2(v_lin1_b[l]),
                       v_lin2_W[l], r2(v_lin2_b[l]))
        xs.append(v)

    out = pl.pallas_call(
        functools.partial(_final_kernel, num_graphs=num_graphs),
        grid=(num_tiles,),
        in_specs=[rowblk(H), rowblk(H), rowblk(H), full((1, npad)),
                  full(u_lin1_W.shape), full((1, H)), full(u_lin2_W.shape),
                  full((1, H)), smem],
        out_specs=pl.BlockSpec((num_graphs, H), lambda t: (0, 0)),
        out_shape=jax.ShapeDtypeStruct((num_graphs, H), jnp.float32),
        scratch_shapes=[pltpu.VMEM((num_graphs, H), jnp.float32)],
        interpret=interpret,
    )(xs[0], xs[1], xs[2], batchT, u_lin1_W, r2(u_lin1_b), u_lin2_W,
      r2(u_lin2_b), jnp.array([num_tiles], jnp.int32))
    return out


def kernel(z, pos, batch, emb1, emb2, emb3, emb4, lin1_W, lin1_b, e_mlp1_W,
           e_mlp1_b, e_mlp2_W, e_mlp2_b, e_lin_W, v_lin1_W, v_lin1_b,
           v_lin2_W, v_lin2_b, u_lin1_W, u_lin1_b, u_lin2_W, u_lin2_b):
    return _forward(z, pos, batch, emb1, emb2, emb3, emb4, lin1_W, lin1_b,
                    e_mlp1_W, e_mlp1_b, e_mlp2_W, e_mlp2_b, e_lin_W, v_lin1_W,
                    v_lin1_b, v_lin2_W, v_lin2_b, u_lin1_W, u_lin1_b, u_lin2_W,
                    u_lin2_b, NUM_GRAPHS)


# speedup vs baseline: 1.6972x; 1.1540x over previous
import functools

import jax
import jax.numpy as jnp
from jax import lax
from jax.experimental import pallas as pl
from jax.experimental.pallas import tpu as pltpu

CUTOFF = 5.0
G = 50
H = 128
NUM_GRAPHS = 256
TI = 128
TIL = 64
TJ = 128


def _embed_kernel(z_ref, emb1_ref, emb2_ref, emb3_ref, emb4_ref, W_ref, b_ref,
                  v_ref):
    z = z_ref[...]

    def onehot(col, k):
        ids = z[:, col][:, None]
        return (ids == lax.broadcasted_iota(jnp.int32, (TI, k), 1)).astype(
            jnp.float32)

    e1 = jnp.dot(onehot(0, 100), emb1_ref[...],
                 preferred_element_type=jnp.float32)
    e2 = jnp.dot(onehot(1, 10), emb2_ref[...],
                 preferred_element_type=jnp.float32)
    e3 = jnp.dot(onehot(2, 10), emb3_ref[...],
                 preferred_element_type=jnp.float32)
    e4 = jnp.dot(onehot(3, 10), emb4_ref[...],
                 preferred_element_type=jnp.float32)
    W = W_ref[...]
    v = (jnp.dot(e1, W[:, 0:H].T, preferred_element_type=jnp.float32)
         + jnp.dot(e2, W[:, H:2 * H].T, preferred_element_type=jnp.float32)
         + jnp.dot(e3, W[:, 2 * H:3 * H].T, preferred_element_type=jnp.float32)
         + jnp.dot(e4, W[:, 3 * H:4 * H].T, preferred_element_type=jnp.float32)
         + b_ref[...])
    v_ref[...] = v


def _layer_kernel(jstart8_ref, jnum_ref, feati_ref, featj_ref, v_ref,
                  elW_ref, W1_ref, b1_ref, W2_ref, b2_ref, v1W_ref, v1b_ref,
                  v2W_ref, v2b_ref, vout_ref):
    t = pl.program_id(0)
    i0 = t * TIL
    iidx = i0 + lax.broadcasted_iota(jnp.int32, (TIL, 1), 0)

    step = CUTOFF / (G - 1)
    coeff = -0.5 / (step * step)
    offset = step * lax.broadcasted_iota(jnp.int32, (1, 1, G), 2).astype(
        jnp.float32)
    W1t = W1_ref[...].T
    W2t = W2_ref[...].T
    b1 = b1_ref[...]
    b2 = b2_ref[...]
    elWt = elW_ref[...].T
    jbase = jstart8_ref[t] * 8
    feati = feati_ref[...]
    fi_pos = feati[:, 0:5]
    fi_b = feati[:, 5:8]

    def body(jt, acc):
        j0 = jbase + jt * TJ
        featj = featj_ref[pl.ds(j0, TJ), :]
        vj = v_ref[pl.ds(j0, TJ), :]
        vlj = jnp.dot(vj, elWt, preferred_element_type=jnp.float32)

        cn = (((1,), (1,)), ((), ()))
        d2 = lax.dot_general(fi_pos, featj[:, 0:5], cn,
                             precision=lax.Precision.HIGHEST,
                             preferred_element_type=jnp.float32)
        db2 = lax.dot_general(fi_b, featj[:, 5:8], cn,
                              precision=lax.Precision.HIGHEST,
                              preferred_element_type=jnp.float32)
        d2 = jnp.maximum(d2, 0.0)
        d = jnp.sqrt(d2)
        jidx = j0 + lax.broadcasted_iota(jnp.int32, (1, TJ), 1)
        mask = (d < CUTOFF) & (db2 < 0.5) & (iidx != jidx)
        C = 0.5 * (jnp.cos(d * (jnp.pi / CUTOFF)) + 1.0)
        scal = jnp.where(mask, C, 0.0)

        d16 = d.astype(jnp.bfloat16)
        de = jnp.exp(jnp.bfloat16(coeff)
                     * (d16[:, :, None] - offset.astype(jnp.bfloat16)) ** 2)
        de_f = de.reshape(TIL * TJ, G)
        h = jnp.maximum(
            jnp.dot(de_f, W1t.astype(jnp.bfloat16),
                    preferred_element_type=jnp.float32) + b1, 0.0)
        wg = jnp.dot(h, W2t, preferred_element_type=jnp.float32) + b2
        wg3 = wg.reshape(TIL, TJ, H)
        contrib = wg3 * (scal[:, :, None] * vlj[None, :, :])
        return acc + jnp.sum(contrib, axis=1)

    acc = lax.fori_loop(0, jnum_ref[t], body,
                        jnp.zeros((TIL, H), jnp.float32))

    vi = v_ref[pl.ds(i0, TIL), :]
    o = jnp.maximum(
        jnp.dot(acc, v1W_ref[...].T, preferred_element_type=jnp.float32)
        + v1b_ref[...], 0.0)
    o = jnp.dot(o, v2W_ref[...].T, preferred_element_type=jnp.float32) \
        + v2b_ref[...]
    vout_ref[...] = vi + o


def _final_kernel(x1_ref, x2_ref, x3_ref, batchT_ref, W1_ref, b1_ref, W2_ref,
                  b2_ref, num_tiles_ref, out_ref, acc_ref, *, num_graphs):
    t = pl.program_id(0)

    @pl.when(t == 0)
    def _():
        acc_ref[...] = jnp.zeros_like(acc_ref)

    W1 = W1_ref[...]
    u = (jnp.dot(x1_ref[...], W1[:, 0:H].T, preferred_element_type=jnp.float32)
         + jnp.dot(x2_ref[...], W1[:, H:2 * H].T,
                   preferred_element_type=jnp.float32)
         + jnp.dot(x3_ref[...], W1[:, 2 * H:3 * H].T,
                   preferred_element_type=jnp.float32)
         + b1_ref[...])
    u = jnp.maximum(u, 0.0)

    bj = batchT_ref[:, pl.ds(t * TI, TI)]
    ohT = (lax.broadcasted_iota(jnp.int32, (num_graphs, TI), 0) == bj).astype(
        jnp.float32)
    acc_ref[...] += jnp.dot(ohT, u, preferred_element_type=jnp.float32)

    @pl.when(t == num_tiles_ref[0] - 1)
    def _():
        out_ref[...] = jnp.maximum(
            jnp.dot(acc_ref[...], W2_ref[...].T,
                    preferred_element_type=jnp.float32) + b2_ref[...], 0.0)


def _forward(z, pos, batch, emb1, emb2, emb3, emb4, lin1_W, lin1_b, e_mlp1_W,
             e_mlp1_b, e_mlp2_W, e_mlp2_b, e_lin_W, v_lin1_W, v_lin1_b,
             v_lin2_W, v_lin2_b, u_lin1_W, u_lin1_b, u_lin2_W, u_lin2_b,
             num_graphs, interpret=False):
    n = pos.shape[0]
    num_tiles = (n + TI - 1) // TI
    npad = num_tiles * TI
    pad = npad - n

    batch = batch.astype(jnp.int32)
    z = z.astype(jnp.int32)
    pos_p = jnp.pad(pos, ((0, pad), (0, 0)))
    batch_p = jnp.pad(batch, (0, pad), constant_values=jnp.int32(2 ** 30))
    z_p = jnp.pad(z, ((0, pad), (0, 0)))
    batchT = batch_p[None, :]

    num_itiles = npad // TIL
    tstarts = jnp.arange(num_itiles, dtype=jnp.int32) * TIL
    row_lo = jnp.clip(tstarts, 0, n - 1)
    row_hi = jnp.clip(tstarts + TIL - 1, 0, n - 1)
    blo = batch[row_lo]
    bhi = batch[row_hi]
    jlo = jnp.searchsorted(batch, blo, side="left").astype(jnp.int32)
    jhi = jnp.searchsorted(batch, bhi, side="right").astype(jnp.int32)
    jstart8 = jlo // 8
    jnum = (jhi - jstart8 * 8 + TJ - 1) // TJ

    x, y, zc = pos_p[:, 0], pos_p[:, 1], pos_p[:, 2]
    n2 = x * x + y * y + zc * zc
    bf = batch_p.astype(jnp.float32)
    one = jnp.ones_like(n2)
    feati = jnp.stack([x, y, zc, n2, one, bf * bf, -2.0 * bf, one], axis=1)
    featj = jnp.stack([-2.0 * x, -2.0 * y, -2.0 * zc, one, n2, one, bf,
                       bf * bf], axis=1)

    npad_j = npad + TJ
    featj = jnp.pad(featj, ((0, TJ), (0, 0)))
    featj = featj.at[npad:, 6].set(jnp.float32(2 ** 30))
    featj = featj.at[npad:, 7].set(jnp.float32(2 ** 60))

    r2 = lambda a: a.reshape(1, -1)

    full = lambda shape: pl.BlockSpec(shape, lambda t: (0,) * len(shape))
    rowblk = lambda w: pl.BlockSpec((TI, w), lambda t: (t, 0))
    smem = pl.BlockSpec(memory_space=pltpu.SMEM)

    v = pl.pallas_call(
        _embed_kernel,
        grid=(num_tiles,),
        in_specs=[rowblk(4), full(emb1.shape), full(emb2.shape),
                  full(emb3.shape), full(emb4.shape), full(lin1_W.shape),
                  full((1, H))],
        out_specs=rowblk(H),
        out_shape=jax.ShapeDtypeStruct((npad, H), jnp.float32),
        interpret=interpret,
    )(z_p, emb1, emb2, emb3, emb4, lin1_W, r2(lin1_b))

    rowblk_l = lambda w: pl.BlockSpec((TIL, w), lambda t: (t, 0))
    layer_call = pl.pallas_call(
        _layer_kernel,
        grid=(num_itiles,),
        in_specs=[smem, smem, rowblk_l(8), full((npad_j, 8)),
                  full((npad_j, H)), full((H, H)),
                  full((H, G)), full((1, H)), full((H, H)), full((1, H)),
                  full((H, H)), full((1, H)), full((H, H)), full((1, H))],
        out_specs=rowblk_l(H),
        out_shape=jax.ShapeDtypeStruct((npad, H), jnp.float32),
        interpret=interpret,
    )

    xs = []
    for l in range(3):
        v_j = jnp.pad(v, ((0, TJ), (0, 0)))
        v = layer_call(jstart8, jnum, feati, featj, v_j, e_lin_W[l],
                       e_mlp1_W[l], r2(e_mlp1_b[l]), e_mlp2_W[l],
                       r2(e_mlp2_b[l]), v_lin1_W[l], r2(v_lin1_b[l]),
                       v_lin2_W[l], r2(v_lin2_b[l]))
        xs.append(v)

    out = pl.pallas_call(
        functools.partial(_final_kernel, num_graphs=num_graphs),
        grid=(num_tiles,),
        in_specs=[rowblk(H), rowblk(H), rowblk(H), full((1, npad)),
                  full(u_lin1_W.shape), full((1, H)), full(u_lin2_W.shape),
                  full((1, H)), smem],
        out_specs=pl.BlockSpec((num_graphs, H), lambda t: (0, 0)),
        out_shape=jax.ShapeDtypeStruct((num_graphs, H), jnp.float32),
        scratch_shapes=[pltpu.VMEM((num_graphs, H), jnp.float32)],
        interpret=interpret,
    )(xs[0], xs[1], xs[2], batchT, u_lin1_W, r2(u_lin1_b), u_lin2_W,
      r2(u_lin2_b), jnp.array([num_tiles], jnp.int32))
    return out


def kernel(z, pos, batch, emb1, emb2, emb3, emb4, lin1_W, lin1_b, e_mlp1_W,
           e_mlp1_b, e_mlp2_W, e_mlp2_b, e_lin_W, v_lin1_W, v_lin1_b,
           v_lin2_W, v_lin2_b, u_lin1_W, u_lin1_b, u_lin2_W, u_lin2_b):
    return _forward(z, pos, batch, emb1, emb2, emb3, emb4, lin1_W, lin1_b,
                    e_mlp1_W, e_mlp1_b, e_mlp2_W, e_mlp2_b, e_lin_W, v_lin1_W,
                    v_lin1_b, v_lin2_W, v_lin2_b, u_lin1_W, u_lin1_b, u_lin2_W,
                    u_lin2_b, NUM_GRAPHS)


# speedup vs baseline: 1.6975x; 1.0002x over previous
import functools

import jax
import jax.numpy as jnp
from jax import lax
from jax.experimental import pallas as pl
from jax.experimental.pallas import tpu as pltpu

CUTOFF = 5.0
G = 50
H = 128
NUM_GRAPHS = 256
TI = 128
TIL = 64
TJ = 128


def _embed_kernel(z_ref, emb1_ref, emb2_ref, emb3_ref, emb4_ref, W_ref, b_ref,
                  v_ref):
    z = z_ref[...]

    def onehot(col, k):
        ids = z[:, col][:, None]
        return (ids == lax.broadcasted_iota(jnp.int32, (TI, k), 1)).astype(
            jnp.float32)

    e1 = jnp.dot(onehot(0, 100), emb1_ref[...],
                 preferred_element_type=jnp.float32)
    e2 = jnp.dot(onehot(1, 10), emb2_ref[...],
                 preferred_element_type=jnp.float32)
    e3 = jnp.dot(onehot(2, 10), emb3_ref[...],
                 preferred_element_type=jnp.float32)
    e4 = jnp.dot(onehot(3, 10), emb4_ref[...],
                 preferred_element_type=jnp.float32)
    W = W_ref[...]
    v = (jnp.dot(e1, W[:, 0:H].T, preferred_element_type=jnp.float32)
         + jnp.dot(e2, W[:, H:2 * H].T, preferred_element_type=jnp.float32)
         + jnp.dot(e3, W[:, 2 * H:3 * H].T, preferred_element_type=jnp.float32)
         + jnp.dot(e4, W[:, 3 * H:4 * H].T, preferred_element_type=jnp.float32)
         + b_ref[...])
    v_ref[...] = v


def _layer_kernel(jstart8_ref, jnum_ref, feati_ref, featj_ref, v_ref,
                  elW_ref, W1_ref, b1_ref, W2_ref, b2_ref, v1W_ref, v1b_ref,
                  v2W_ref, v2b_ref, vout_ref):
    t = pl.program_id(0)
    i0 = t * TIL
    iidx = i0 + lax.broadcasted_iota(jnp.int32, (TIL, 1), 0)

    step = CUTOFF / (G - 1)
    coeff = -0.5 / (step * step)
    offset = step * lax.broadcasted_iota(jnp.int32, (1, 1, G), 2).astype(
        jnp.float32)
    W1t = W1_ref[...].T
    W2t = W2_ref[...].T
    b1 = b1_ref[...]
    b2 = b2_ref[...]
    elWt = elW_ref[...].T
    jbase = jstart8_ref[t] * 8
    feati = feati_ref[...]
    fi_pos = feati[:, 0:5]
    fi_b = feati[:, 5:8]

    def body(jt, acc):
        j0 = jbase + jt * TJ
        featj = featj_ref[pl.ds(j0, TJ), :]
        vj = v_ref[pl.ds(j0, TJ), :]
        vlj = jnp.dot(vj, elWt, preferred_element_type=jnp.float32)

        cn = (((1,), (1,)), ((), ()))
        d2 = lax.dot_general(fi_pos, featj[:, 0:5], cn,
                             precision=lax.Precision.HIGHEST,
                             preferred_element_type=jnp.float32)
        db2 = lax.dot_general(fi_b, featj[:, 5:8], cn,
                              precision=lax.Precision.HIGHEST,
                              preferred_element_type=jnp.float32)
        d2 = jnp.maximum(d2, 0.0)
        d = jnp.sqrt(d2)
        jidx = j0 + lax.broadcasted_iota(jnp.int32, (1, TJ), 1)
        mask = (d < CUTOFF) & (db2 < 0.5) & (iidx != jidx)
        C = 0.5 * (jnp.cos(d * (jnp.pi / CUTOFF)) + 1.0)
        scal = jnp.where(mask, C, 0.0)

        d16 = d.astype(jnp.bfloat16)
        de = jnp.exp(jnp.bfloat16(coeff)
                     * (d16[:, :, None] - offset.astype(jnp.bfloat16)) ** 2)
        de_f = de.reshape(TIL * TJ, G)
        h = jnp.maximum(
            jnp.dot(de_f, W1t.astype(jnp.bfloat16),
                    preferred_element_type=jnp.float32) + b1, 0.0)
        wg = jnp.dot(h, W2t, preferred_element_type=jnp.float32) + b2
        wg3 = wg.reshape(TIL, TJ, H)
        contrib = wg3 * (scal[:, :, None] * vlj[None, :, :])
        return acc + jnp.sum(contrib, axis=1)

    acc = lax.fori_loop(0, jnum_ref[t], body,
                        jnp.zeros((TIL, H), jnp.float32))

    vi = v_ref[pl.ds(i0, TIL), :]
    o = jnp.maximum(
        jnp.dot(acc, v1W_ref[...].T, preferred_element_type=jnp.float32)
        + v1b_ref[...], 0.0)
    o = jnp.dot(o, v2W_ref[...].T, preferred_element_type=jnp.float32) \
        + v2b_ref[...]
    vout_ref[...] = vi + o


def _final_kernel(x1_ref, x2_ref, x3_ref, batchT_ref, W1_ref, b1_ref, W2_ref,
                  b2_ref, num_tiles_ref, out_ref, acc_ref, *, num_graphs):
    t = pl.program_id(0)

    @pl.when(t == 0)
    def _():
        acc_ref[...] = jnp.zeros_like(acc_ref)

    W1 = W1_ref[...]
    u = (jnp.dot(x1_ref[...], W1[:, 0:H].T, preferred_element_type=jnp.float32)
         + jnp.dot(x2_ref[...], W1[:, H:2 * H].T,
                   preferred_element_type=jnp.float32)
         + jnp.dot(x3_ref[...], W1[:, 2 * H:3 * H].T,
                   preferred_element_type=jnp.float32)
         + b1_ref[...])
    u = jnp.maximum(u, 0.0)

    bj = batchT_ref[:, pl.ds(t * TI, TI)]
    ohT = (lax.broadcasted_iota(jnp.int32, (num_graphs, TI), 0) == bj).astype(
        jnp.float32)
    acc_ref[...] += jnp.dot(ohT, u, preferred_element_type=jnp.float32)

    @pl.when(t == num_tiles_ref[0] - 1)
    def _():
        out_ref[...] = jnp.maximum(
            jnp.dot(acc_ref[...], W2_ref[...].T,
                    preferred_element_type=jnp.float32) + b2_ref[...], 0.0)


def _forward(z, pos, batch, emb1, emb2, emb3, emb4, lin1_W, lin1_b, e_mlp1_W,
             e_mlp1_b, e_mlp2_W, e_mlp2_b, e_lin_W, v_lin1_W, v_lin1_b,
             v_lin2_W, v_lin2_b, u_lin1_W, u_lin1_b, u_lin2_W, u_lin2_b,
             num_graphs, interpret=False):
    n = pos.shape[0]
    num_tiles = (n + TI - 1) // TI
    npad = num_tiles * TI
    pad = npad - n

    batch = batch.astype(jnp.int32)
    z = z.astype(jnp.int32)
    pos_p = jnp.pad(pos, ((0, pad), (0, 0)))
    batch_p = jnp.pad(batch, (0, pad), constant_values=jnp.int32(2 ** 30))
    z_p = jnp.pad(z, ((0, pad), (0, 0)))
    batchT = batch_p[None, :]

    num_itiles = npad // TIL
    tstarts = jnp.arange(num_itiles, dtype=jnp.int32) * TIL
    row_lo = jnp.clip(tstarts, 0, n - 1)
    row_hi = jnp.clip(tstarts + TIL - 1, 0, n - 1)
    blo = batch[row_lo]
    bhi = batch[row_hi]
    jlo = jnp.searchsorted(batch, blo, side="left").astype(jnp.int32)
    jhi = jnp.searchsorted(batch, bhi, side="right").astype(jnp.int32)
    jstart8 = jlo // 8
    jnum = (jhi - jstart8 * 8 + TJ - 1) // TJ

    x, y, zc = pos_p[:, 0], pos_p[:, 1], pos_p[:, 2]
    n2 = x * x + y * y + zc * zc
    bf = batch_p.astype(jnp.float32)
    one = jnp.ones_like(n2)
    feati = jnp.stack([x, y, zc, n2, one, bf * bf, -2.0 * bf, one], axis=1)
    featj = jnp.stack([-2.0 * x, -2.0 * y, -2.0 * zc, one, n2, one, bf,
                       bf * bf], axis=1)

    npad_j = npad + TJ
    featj = jnp.pad(featj, ((0, TJ), (0, 0)))
    featj = featj.at[npad:, 6].set(jnp.float32(2 ** 30))
    featj = featj.at[npad:, 7].set(jnp.float32(2 ** 60))

    r2 = lambda a: a.reshape(1, -1)

    full = lambda shape: pl.BlockSpec(shape, lambda t: (0,) * len(shape))
    rowblk = lambda w: pl.BlockSpec((TI, w), lambda t: (t, 0))
    smem = pl.BlockSpec(memory_space=pltpu.SMEM)

    v = pl.pallas_call(
        _embed_kernel,
        grid=(num_tiles,),
        in_specs=[rowblk(4), full(emb1.shape), full(emb2.shape),
                  full(emb3.shape), full(emb4.shape), full(lin1_W.shape),
                  full((1, H))],
        out_specs=rowblk(H),
        out_shape=jax.ShapeDtypeStruct((npad, H), jnp.float32),
        interpret=interpret,
    )(z_p, emb1, emb2, emb3, emb4, lin1_W, r2(lin1_b))

    rowblk_l = lambda w: pl.BlockSpec((TIL, w), lambda t: (t, 0))
    layer_call = pl.pallas_call(
        _layer_kernel,
        grid=(num_itiles,),
        in_specs=[smem, smem, rowblk_l(8), full((npad_j, 8)),
                  full((npad_j, H)), full((H, H)),
                  full((H, G)), full((1, H)), full((H, H)), full((1, H)),
                  full((H, H)), full((1, H)), full((H, H)), full((1, H))],
        out_specs=rowblk_l(H),
        out_shape=jax.ShapeDtypeStruct((npad, H), jnp.float32),
        compiler_params=pltpu.CompilerParams(
            dimension_semantics=("parallel",)),
        interpret=interpret,
    )

    xs = []
    for l in range(3):
        v_j = jnp.pad(v, ((0, TJ), (0, 0)))
        v = layer_call(jstart8, jnum, feati, featj, v_j, e_lin_W[l],
                       e_mlp1_W[l], r2(e_mlp1_b[l]), e_mlp2_W[l],
                       r2(e_mlp2_b[l]), v_lin1_W[l], r2(v_lin1_b[l]),
                       v_lin2_W[l], r2(v_lin2_b[l]))
        xs.append(v)

    out = pl.pallas_call(
        functools.partial(_final_kernel, num_graphs=num_graphs),
        grid=(num_tiles,),
        in_specs=[rowblk(H), rowblk(H), rowblk(H), full((1, npad)),
                  full(u_lin1_W.shape), full((1, H)), full(u_lin2_W.shape),
                  full((1, H)), smem],
        out_specs=pl.BlockSpec((num_graphs, H), lambda t: (0, 0)),
        out_shape=jax.ShapeDtypeStruct((num_graphs, H), jnp.float32),
        scratch_shapes=[pltpu.VMEM((num_graphs, H), jnp.float32)],
        interpret=interpret,
    )(xs[0], xs[1], xs[2], batchT, u_lin1_W, r2(u_lin1_b), u_lin2_W,
      r2(u_lin2_b), jnp.array([num_tiles], jnp.int32))
    return out


def kernel(z, pos, batch, emb1, emb2, emb3, emb4, lin1_W, lin1_b, e_mlp1_W,
           e_mlp1_b, e_mlp2_W, e_mlp2_b, e_lin_W, v_lin1_W, v_lin1_b,
           v_lin2_W, v_lin2_b, u_lin1_W, u_lin1_b, u_lin2_W, u_lin2_b):
    return _forward(z, pos, batch, emb1, emb2, emb3, emb4, lin1_W, lin1_b,
                    e_mlp1_W, e_mlp1_b, e_mlp2_W, e_mlp2_b, e_lin_W, v_lin1_W,
                    v_lin1_b, v_lin2_W, v_lin2_b, u_lin1_W, u_lin1_b, u_lin2_W,
                    u_lin2_b, NUM_GRAPHS)
